# baseline (speedup 1.0000x reference)
import functools

import jax
import jax.numpy as jnp
from jax import lax
from jax.experimental import pallas as pl
from jax.experimental.pallas import tpu as pltpu
from jax.experimental.pallas import tpu_sc as plsc

_NPTS = 16384

_SC_NC = 2
_SC_NS = 16
_SC_NW = _SC_NC * _SC_NS


def _sc_gather(table, idx):
    V, D = table.shape
    B = idx.shape[0]
    assert D % 16 == 0 and B % (8 * _SC_NW) == 0, (V, D, B)
    b_per_w = B // _SC_NW
    chunk = b_per_w
    while chunk * D * 4 > 240 * 1024:
        chunk //= 2
    n_chunks = b_per_w // chunk
    mesh = plsc.VectorSubcoreMesh(core_axis_name="c", subcore_axis_name="s")

    @functools.partial(
        pl.kernel,
        mesh=mesh,
        compiler_params=pltpu.CompilerParams(use_tc_tiling_on_sc=False),
        out_type=jax.ShapeDtypeStruct((B, D), jnp.float32),
        scratch_types=[
            pltpu.VMEM((chunk,), jnp.int32),
            pltpu.VMEM((chunk, D), jnp.float32),
            pltpu.SemaphoreType.DMA,
        ],
    )
    def k(table_hbm, idx_hbm, out_hbm, idx_v, rows_v, sem):
        wid = lax.axis_index("s") * _SC_NC + lax.axis_index("c")
        base = wid * b_per_w

        def step(j, carry):
            off = base + j * chunk
            pltpu.sync_copy(idx_hbm.at[pl.ds(off, chunk)], idx_v)
            pltpu.async_copy(table_hbm.at[idx_v], rows_v, sem).wait()
            pltpu.sync_copy(rows_v, out_hbm.at[pl.ds(off, chunk)])
            return carry

        lax.fori_loop(0, n_chunks, step, 0)

    return k(table, idx)


def _grouped_gather(xyz, fT, nn):
    Bb, N, _ = xyz.shape
    C = fT.shape[-1]
    Q, k = nn.shape[1], nn.shape[2]
    D = 3 + C
    Dpad = ((D + 15) // 16) * 16
    table = jnp.concatenate([xyz, fT], axis=-1).reshape(Bb * N, D)
    table = jnp.pad(table, ((0, 0), (0, Dpad - D)))
    offs = (jnp.arange(Bb, dtype=jnp.int32) * N)[:, None, None]
    flat_idx = (nn + offs).reshape(Bb * Q * k)
    rows = _sc_gather(table, flat_idx)
    return rows.reshape(Bb, Q, k, Dpad)[..., :D]


def _fps_body(npoint, n, xr_ref, yr_ref, zr_ref, out_ref):
    R, C = xr_ref.shape[1], xr_ref.shape[2]
    xr = xr_ref[0]
    yr = yr_ref[0]
    zr = zr_ref[0]
    gidx = (jax.lax.broadcasted_iota(jnp.int32, (R, C), 0) * C
            + jax.lax.broadcasted_iota(jnp.int32, (R, C), 1))

    def body(i, state):
        dists, farthest = state
        sel = gidx == farthest
        cx = jnp.sum(jnp.where(sel, xr, 0.0))
        cy = jnp.sum(jnp.where(sel, yr, 0.0))
        cz = jnp.sum(jnp.where(sel, zr, 0.0))
        cvec = jnp.concatenate(
            [cx.reshape(1, 1), cy.reshape(1, 1), cz.reshape(1, 1)], axis=1)
        out_ref[0, pl.ds(i, 1), :] = cvec
        dx = xr - cx
        dy = yr - cy
        dz = zr - cz
        d = (dx * dx + dy * dy) + dz * dz
        dists = jnp.minimum(dists, d)
        m = jnp.max(dists)
        cand = jnp.where(dists == m, gidx, n)
        farthest = jnp.min(cand).astype(jnp.int32)
        return (dists, farthest)

    init = (jnp.full((R, C), 1e10, jnp.float32), jnp.int32(0))
    jax.lax.fori_loop(0, npoint, body, init)


def _fps_new_xyz(xyz, npoint):
    Bb, N, _ = xyz.shape
    R = 8
    C = N // R
    xr = xyz[:, :, 0].reshape(Bb, R, C)
    yr = xyz[:, :, 1].reshape(Bb, R, C)
    zr = xyz[:, :, 2].reshape(Bb, R, C)
    return pl.pallas_call(
        functools.partial(_fps_body, npoint, N),
        grid=(Bb,),
        in_specs=[
            pl.BlockSpec((1, R, C), lambda b: (b, 0, 0)),
            pl.BlockSpec((1, R, C), lambda b: (b, 0, 0)),
            pl.BlockSpec((1, R, C), lambda b: (b, 0, 0)),
        ],
        out_specs=pl.BlockSpec((1, npoint, 3), lambda b: (b, 0, 0)),
        out_shape=jax.ShapeDtypeStruct((Bb, npoint, 3), jnp.float32),
    )(xr, yr, zr)


def _knn(query, points, k, chunk=512):
    query = jax.lax.stop_gradient(query)
    points = jax.lax.stop_gradient(points)
    Q = query.shape[1]
    outs = []
    for s in range(0, Q, chunk):
        q = query[:, s:s + chunk]
        d = jnp.sum((q[:, :, None, :] - points[:, None, :, :]) ** 2, axis=-1)
        outs.append(jax.lax.top_k(-d, k)[1])
    return jnp.concatenate(outs, axis=1)


def _gather_points(arr, idx):
    Bb, Q, k = idx.shape
    D = arr.shape[-1]
    flat = jnp.broadcast_to(idx.reshape(Bb, Q * k, 1), (Bb, Q * k, D))
    return jnp.take_along_axis(arr, flat, axis=1).reshape(Bb, Q, k, D)


def _point_conv(xyz, features, W, b, npoint, k, ctr_xyz=None):
    if ctr_xyz is None:
        new_xyz = _fps_new_xyz(jax.lax.stop_gradient(xyz), npoint)
    else:
        new_xyz = ctr_xyz
    nn = _knn(new_xyz, xyz, k)
    fT = features.transpose(0, 2, 1)
    g = _grouped_gather(xyz, fT, nn)
    g = jnp.concatenate([g[..., :3] - new_xyz[:, :, None, :], g[..., 3:]],
                        axis=-1)
    h = jax.nn.relu(g @ W + b)
    new_f = jnp.max(h, axis=2).transpose(0, 2, 1)
    return new_xyz, new_f


def _vote_kernel(f_ref, wm_ref, bm_ref, wo_ref, bo_ref, h_ref, off_ref):
    fT = f_ref[0]
    h = jnp.maximum(fT @ wm_ref[...] + bm_ref[...][None, :], 0.0)
    off = h @ wo_ref[...] + bo_ref[...][None, :]
    h_ref[0] = h
    off_ref[0] = off


def _vote_layer(xyz, features, Wm, bm, Wo, bo, max_range):
    Bb, C, Q = features.shape
    fT = features.transpose(0, 2, 1)
    H = Wm.shape[1]
    O = Wo.shape[1]
    h, off = pl.pallas_call(
        _vote_kernel,
        grid=(Bb,),
        in_specs=[
            pl.BlockSpec((1, Q, C), lambda b: (b, 0, 0)),
            pl.BlockSpec((C, H), lambda b: (0, 0)),
            pl.BlockSpec((H,), lambda b: (0,)),
            pl.BlockSpec((H, O), lambda b: (0, 0)),
            pl.BlockSpec((O,), lambda b: (0,)),
        ],
        out_specs=[
            pl.BlockSpec((1, Q, H), lambda b: (b, 0, 0)),
            pl.BlockSpec((1, Q, O), lambda b: (b, 0, 0)),
        ],
        out_shape=[
            jax.ShapeDtypeStruct((Bb, Q, H), jnp.float32),
            jax.ShapeDtypeStruct((Bb, Q, O), jnp.float32),
        ],
    )(fT, Wm, bm, Wo, bo)
    limited = jnp.clip(off, -max_range, max_range)
    new_xyz = xyz + limited
    return new_xyz, h.transpose(0, 2, 1), limited


def _range_encoded(xyz, feature):
    R = 70.4
    rng = jnp.linalg.norm(xyz, axis=2)
    color = feature[:, 1:, :]
    scale = (rng / (R * 255.0))[:, None, :]
    return jnp.concatenate([feature[:, 0:1, :], color * scale], axis=1)


def kernel(points, batch_size, sa0_W, sa0_b, sa1_W, sa1_b, sa2_W, sa2_b,
           vote_W, vote_b, vote_off_W, vote_off_b, sa4_W, sa4_b):
    bs = points.shape[0] // _NPTS
    xyz = points[:, 1:4].reshape(bs, -1, 3)
    xyz = xyz + jnp.zeros((), xyz.dtype) * batch_size
    feats = points[:, 4:].reshape(bs, -1, 4).transpose(0, 2, 1)
    feats = _range_encoded(xyz, feats)
    x0, f0 = _point_conv(xyz, feats, sa0_W, sa0_b, 4096, 32)
    x1, f1 = _point_conv(x0, f0, sa1_W, sa1_b, 1024, 32)
    x2, f2 = _point_conv(x1, f1, sa2_W, sa2_b, 512, 32)
    max_range = jnp.array([3.0, 3.0, 2.0], dtype=jnp.float32)
    x3, f3, ctr_offsets = _vote_layer(x2, f2, vote_W, vote_b,
                                      vote_off_W, vote_off_b, max_range)
    x4, f4 = _point_conv(x2, f2, sa4_W, sa4_b, 256, 32, ctr_xyz=x3)
    center_features = f4.transpose(0, 2, 1).reshape(-1, f4.shape[1])
    return center_features


# speedup vs baseline: 3.6024x; 3.6024x over previous
import functools

import jax
import jax.numpy as jnp
from jax import lax
from jax.experimental import pallas as pl
from jax.experimental.pallas import tpu as pltpu
from jax.experimental.pallas import tpu_sc as plsc

_NPTS = 16384

_SC_NC = 2
_SC_NS = 16
_SC_NW = _SC_NC * _SC_NS


def _sc_gather(table, idx):
    V, D = table.shape
    B = idx.shape[0]
    assert D % 16 == 0 and B % (8 * _SC_NW) == 0, (V, D, B)
    b_per_w = B // _SC_NW
    chunk = b_per_w
    while chunk * D * 4 > 240 * 1024:
        chunk //= 2
    n_chunks = b_per_w // chunk
    mesh = plsc.VectorSubcoreMesh(core_axis_name="c", subcore_axis_name="s")

    @functools.partial(
        pl.kernel,
        mesh=mesh,
        compiler_params=pltpu.CompilerParams(use_tc_tiling_on_sc=False),
        out_type=jax.ShapeDtypeStruct((B, D), jnp.float32),
        scratch_types=[
            pltpu.VMEM((chunk,), jnp.int32),
            pltpu.VMEM((chunk, D), jnp.float32),
            pltpu.SemaphoreType.DMA,
        ],
    )
    def k(table_hbm, idx_hbm, out_hbm, idx_v, rows_v, sem):
        wid = lax.axis_index("s") * _SC_NC + lax.axis_index("c")
        base = wid * b_per_w

        def step(j, carry):
            off = base + j * chunk
            pltpu.sync_copy(idx_hbm.at[pl.ds(off, chunk)], idx_v)
            pltpu.async_copy(table_hbm.at[idx_v], rows_v, sem).wait()
            pltpu.sync_copy(rows_v, out_hbm.at[pl.ds(off, chunk)])
            return carry

        lax.fori_loop(0, n_chunks, step, 0)

    return k(table, idx)


def _grouped_gather(xyz, fT, nn):
    Bb, N, _ = xyz.shape
    C = fT.shape[-1]
    Q, k = nn.shape[1], nn.shape[2]
    D = 3 + C
    Dpad = ((D + 15) // 16) * 16
    table = jnp.concatenate([xyz, fT], axis=-1).reshape(Bb * N, D)
    table = jnp.pad(table, ((0, 0), (0, Dpad - D)))
    offs = (jnp.arange(Bb, dtype=jnp.int32) * N)[:, None, None]
    flat_idx = (nn + offs).reshape(Bb * Q * k)
    rows = _sc_gather(table, flat_idx)
    return rows.reshape(Bb, Q, k, Dpad)[..., :D]


def _fps_body(npoint, n, xr_ref, yr_ref, zr_ref, out_ref):
    R, C = xr_ref.shape[1], xr_ref.shape[2]
    xr = xr_ref[0]
    yr = yr_ref[0]
    zr = zr_ref[0]
    gidx = (jax.lax.broadcasted_iota(jnp.int32, (R, C), 0) * C
            + jax.lax.broadcasted_iota(jnp.int32, (R, C), 1))

    def body(i, state):
        dists, farthest = state
        sel = gidx == farthest
        cx = jnp.sum(jnp.where(sel, xr, 0.0))
        cy = jnp.sum(jnp.where(sel, yr, 0.0))
        cz = jnp.sum(jnp.where(sel, zr, 0.0))
        cvec = jnp.concatenate(
            [cx.reshape(1, 1), cy.reshape(1, 1), cz.reshape(1, 1)], axis=1)
        out_ref[0, pl.ds(i, 1), :] = cvec
        dx = xr - cx
        dy = yr - cy
        dz = zr - cz
        d = (dx * dx + dy * dy) + dz * dz
        dists = jnp.minimum(dists, d)
        m = jnp.max(dists)
        cand = jnp.where(dists == m, gidx, n)
        farthest = jnp.min(cand).astype(jnp.int32)
        return (dists, farthest)

    init = (jnp.full((R, C), 1e10, jnp.float32), jnp.int32(0))
    jax.lax.fori_loop(0, npoint, body, init)


def _fps_new_xyz(xyz, npoint):
    Bb, N, _ = xyz.shape
    R = 8
    C = N // R
    xr = xyz[:, :, 0].reshape(Bb, R, C)
    yr = xyz[:, :, 1].reshape(Bb, R, C)
    zr = xyz[:, :, 2].reshape(Bb, R, C)
    return pl.pallas_call(
        functools.partial(_fps_body, npoint, N),
        grid=(Bb,),
        in_specs=[
            pl.BlockSpec((1, R, C), lambda b: (b, 0, 0)),
            pl.BlockSpec((1, R, C), lambda b: (b, 0, 0)),
            pl.BlockSpec((1, R, C), lambda b: (b, 0, 0)),
        ],
        out_specs=pl.BlockSpec((1, npoint, 3), lambda b: (b, 0, 0)),
        out_shape=jax.ShapeDtypeStruct((Bb, npoint, 3), jnp.float32),
    )(xr, yr, zr)


def _knn_body(k, n, q_ref, px_ref, py_ref, pz_ref, out_ref, d_ref):
    bq = q_ref.shape[1]
    qx = q_ref[0, :, 0:1]
    qy = q_ref[0, :, 1:2]
    qz = q_ref[0, :, 2:3]
    px = px_ref[0]
    py = py_ref[0]
    pz = pz_ref[0]
    dx = qx - px
    dy = qy - py
    dz = qz - pz
    d_ref[...] = (dx * dx + dy * dy) + dz * dz
    gidx = jax.lax.broadcasted_iota(jnp.int32, (bq, n), 1)
    kiota = jax.lax.broadcasted_iota(jnp.int32, (bq, k), 1)
    inf = jnp.float32(float("inf"))

    def body(i, acc):
        d = d_ref[...]
        m = jnp.min(d, axis=1, keepdims=True)
        cand = jnp.where(d == m, gidx, n)
        idx = jnp.min(cand, axis=1, keepdims=True)
        acc = jnp.where(kiota == i, idx, acc)
        d_ref[...] = jnp.where(gidx == idx, inf, d)
        return acc

    out_ref[0] = jax.lax.fori_loop(
        0, k, body, jnp.zeros((bq, k), jnp.int32))


def _knn(query, points, k, bq=32):
    Bb, Q, _ = query.shape
    N = points.shape[1]
    px = points[:, :, 0].reshape(Bb, 1, N)
    py = points[:, :, 1].reshape(Bb, 1, N)
    pz = points[:, :, 2].reshape(Bb, 1, N)
    return pl.pallas_call(
        functools.partial(_knn_body, k, N),
        grid=(Bb, Q // bq),
        in_specs=[
            pl.BlockSpec((1, bq, 3), lambda b, q: (b, q, 0)),
            pl.BlockSpec((1, 1, N), lambda b, q: (b, 0, 0)),
            pl.BlockSpec((1, 1, N), lambda b, q: (b, 0, 0)),
            pl.BlockSpec((1, 1, N), lambda b, q: (b, 0, 0)),
        ],
        out_specs=pl.BlockSpec((1, bq, k), lambda b, q: (b, q, 0)),
        out_shape=jax.ShapeDtypeStruct((Bb, Q, k), jnp.int32),
        scratch_shapes=[pltpu.VMEM((bq, N), jnp.float32)],
    )(query, px, py, pz)


def _gather_points(arr, idx):
    Bb, Q, k = idx.shape
    D = arr.shape[-1]
    flat = jnp.broadcast_to(idx.reshape(Bb, Q * k, 1), (Bb, Q * k, D))
    return jnp.take_along_axis(arr, flat, axis=1).reshape(Bb, Q, k, D)


def _point_conv(xyz, features, W, b, npoint, k, ctr_xyz=None):
    if ctr_xyz is None:
        new_xyz = _fps_new_xyz(jax.lax.stop_gradient(xyz), npoint)
    else:
        new_xyz = ctr_xyz
    nn = _knn(jax.lax.stop_gradient(new_xyz), jax.lax.stop_gradient(xyz), k)
    fT = features.transpose(0, 2, 1)
    g = _grouped_gather(xyz, fT, nn)
    g = jnp.concatenate([g[..., :3] - new_xyz[:, :, None, :], g[..., 3:]],
                        axis=-1)
    h = jax.nn.relu(g @ W + b)
    new_f = jnp.max(h, axis=2).transpose(0, 2, 1)
    return new_xyz, new_f


def _vote_kernel(f_ref, wm_ref, bm_ref, wo_ref, bo_ref, h_ref, off_ref):
    fT = f_ref[0]
    h = jnp.maximum(fT @ wm_ref[...] + bm_ref[...][None, :], 0.0)
    off = h @ wo_ref[...] + bo_ref[...][None, :]
    h_ref[0] = h
    off_ref[0] = off


def _vote_layer(xyz, features, Wm, bm, Wo, bo, max_range):
    Bb, C, Q = features.shape
    fT = features.transpose(0, 2, 1)
    H = Wm.shape[1]
    O = Wo.shape[1]
    h, off = pl.pallas_call(
        _vote_kernel,
        grid=(Bb,),
        in_specs=[
            pl.BlockSpec((1, Q, C), lambda b: (b, 0, 0)),
            pl.BlockSpec((C, H), lambda b: (0, 0)),
            pl.BlockSpec((H,), lambda b: (0,)),
            pl.BlockSpec((H, O), lambda b: (0, 0)),
            pl.BlockSpec((O,), lambda b: (0,)),
        ],
        out_specs=[
            pl.BlockSpec((1, Q, H), lambda b: (b, 0, 0)),
            pl.BlockSpec((1, Q, O), lambda b: (b, 0, 0)),
        ],
        out_shape=[
            jax.ShapeDtypeStruct((Bb, Q, H), jnp.float32),
            jax.ShapeDtypeStruct((Bb, Q, O), jnp.float32),
        ],
    )(fT, Wm, bm, Wo, bo)
    limited = jnp.clip(off, -max_range, max_range)
    new_xyz = xyz + limited
    return new_xyz, h.transpose(0, 2, 1), limited


def _range_encoded(xyz, feature):
    R = 70.4
    rng = jnp.linalg.norm(xyz, axis=2)
    color = feature[:, 1:, :]
    scale = (rng / (R * 255.0))[:, None, :]
    return jnp.concatenate([feature[:, 0:1, :], color * scale], axis=1)


def kernel(points, batch_size, sa0_W, sa0_b, sa1_W, sa1_b, sa2_W, sa2_b,
           vote_W, vote_b, vote_off_W, vote_off_b, sa4_W, sa4_b):
    bs = points.shape[0] // _NPTS
    xyz = points[:, 1:4].reshape(bs, -1, 3)
    xyz = xyz + jnp.zeros((), xyz.dtype) * batch_size
    feats = points[:, 4:].reshape(bs, -1, 4).transpose(0, 2, 1)
    feats = _range_encoded(xyz, feats)
    x0, f0 = _point_conv(xyz, feats, sa0_W, sa0_b, 4096, 32)
    x1, f1 = _point_conv(x0, f0, sa1_W, sa1_b, 1024, 32)
    x2, f2 = _point_conv(x1, f1, sa2_W, sa2_b, 512, 32)
    max_range = jnp.array([3.0, 3.0, 2.0], dtype=jnp.float32)
    x3, f3, ctr_offsets = _vote_layer(x2, f2, vote_W, vote_b,
                                      vote_off_W, vote_off_b, max_range)
    x4, f4 = _point_conv(x2, f2, sa4_W, sa4_b, 256, 32, ctr_xyz=x3)
    center_features = f4.transpose(0, 2, 1).reshape(-1, f4.shape[1])
    return center_features
